# baseline (speedup 1.0000x reference)
import functools

import jax
import jax.numpy as jnp
from jax import lax
from jax.experimental import pallas as pl
from jax.experimental.pallas import tpu as pltpu
from jax.experimental.pallas import tpu_sc as plsc

_NS = 16
_NC = 2
_CH = 128
_LANES = 16
_RW = 144


def _pre_body(x_ref, w_ref, a_ref, hx_ref, d4_ref):
    h = jnp.dot(x_ref[...], w_ref[...], preferred_element_type=jnp.float32)
    sd = jnp.dot(h, a_ref[...], preferred_element_type=jnp.float32)
    pad = jnp.zeros((h.shape[0], _RW - h.shape[1] - sd.shape[1]), jnp.float32)
    hx_ref[...] = jnp.concatenate([h, sd, pad], axis=1)
    d4_ref[...] = sd[:, 4:8]


def _post_body(acc_ref, hx_ref, sexp_ref, bias_ref, out_ref):
    num = acc_ref[:, 0:128]
    den = acc_ref[:, 128:132]
    h = hx_ref[:, 0:128]
    s = hx_ref[:, 128:132]
    d = hx_ref[:, 132:136]
    a = s + d
    ws = jnp.exp(jnp.maximum(a, 0.2 * a))
    den4 = den + ws
    wsb = jnp.dot(ws, sexp_ref[...], preferred_element_type=jnp.float32)
    denb = jnp.dot(den4, sexp_ref[...], preferred_element_type=jnp.float32)
    out_ref[...] = (num + wsb * h) / (denb + 1e-16) + bias_ref[...]


def _make_sc_kernel(n, t, d, e, nbatch):
    ept = e // _NS
    nch = ept // _CH
    rows_per_tile = t // _NS
    bpc = nbatch // _NC
    mesh = plsc.VectorSubcoreMesh(core_axis_name="c", subcore_axis_name="s")

    @functools.partial(
        pl.kernel,
        out_type=jax.ShapeDtypeStruct((n, _RW), jnp.float32),
        mesh=mesh,
        scratch_types=(
            pltpu.VMEM((t * 4,), jnp.float32),
            pltpu.VMEM((nch, _CH), jnp.int32),
            pltpu.VMEM((nch, _CH), jnp.int32),
            pltpu.VMEM((_CH, _RW), jnp.float32),
            pltpu.VMEM((_CH, _RW), jnp.float32),
            pltpu.VMEM((_CH, _RW), jnp.float32),
            pltpu.VMEM_SHARED((t, _RW), jnp.float32),
            pltpu.SemaphoreType.DMA,
            pltpu.SemaphoreType.DMA,
            pltpu.SemaphoreType.DMA,
            pltpu.SemaphoreType.DMA,
            pltpu.SemaphoreType.DMA,
            pltpu.SemaphoreType.DMA,
        ),
        compiler_params=pltpu.CompilerParams(
            needs_layout_passes=False, use_tc_tiling_on_sc=False),
    )
    def sc_kernel(srcm_hbm, dstm_hbm, hx_hbm, dflat_hbm, acc_hbm,
                  sdv, srcg_v, dst_v, hb0, hb1, hb2, acc,
                  gs0, gs1, gs2, ss0, ss1, ss2):
        cid = lax.axis_index("c")
        sid = lax.axis_index("s")
        row0 = sid * nch
        r0 = sid * rows_per_tile
        ilane = lax.iota(jnp.int32, _LANES)
        izero = ilane * 0
        zero16 = izero.astype(jnp.float32)

        pltpu.sync_copy(srcm_hbm.at[pl.ds(row0, nch)], srcg_v)
        pltpu.sync_copy(dstm_hbm.at[pl.ds(row0, nch)], dst_v)
        hbufs = (hb0, hb1, hb2)
        gsems = (gs0, gs1, gs2)
        ssems = (ss0, ss1, ss2)

        def _batch(bi, bcarry):
            b = cid * bpc + bi
            base = b * t

            def _z_hb(r, carry):
                for v in range(_RW // _LANES):
                    hb0[r, pl.ds(v * _LANES, _LANES)] = zero16
                return carry
            lax.fori_loop(0, _CH, _z_hb, 0)
            for half in range(rows_per_tile // _CH):
                pltpu.sync_copy(hb0, acc.at[pl.ds(r0 + half * _CH, _CH)])

            pltpu.sync_copy(dflat_hbm.at[pl.ds(base * 4, t * 4)], sdv)

            delta = jnp.where(bi == 0, base, t)
            def _off(r, carry):
                for g in range(_CH // _LANES):
                    sl = pl.ds(g * _LANES, _LANES)
                    srcg_v[r, sl] = srcg_v[r, sl] + delta
                return carry
            lax.fori_loop(0, nch, _off, 0)

            plsc.subcore_barrier()

            def _compute(hbuf, ci):
                @plsc.parallel_loop(0, _CH // _LANES)
                def _grp(g):
                    rows = g * _LANES + ilane
                    d16 = dst_v[ci, pl.ds(g * _LANES, _LANES)]
                    ws = []
                    for h4 in range(4):
                        dv = plsc.load_gather(sdv, [d16 * 4 + h4])
                        sv = plsc.load_gather(hbuf, [rows, izero + (128 + h4)])
                        aa = sv + dv
                        w = jnp.exp(jnp.maximum(aa, 0.2 * aa))
                        plsc.store_scatter(hbuf, [rows, izero + (128 + h4)], w)
                        ws.append(w)
                    for h4 in range(4):
                        w = ws[h4]

                        def _cols(jj, jc, _w=w, _rows=rows):
                            for dj in range(16):
                                jcol = izero + (jj * 16 + dj)
                                col = plsc.load_gather(hbuf, [_rows, jcol])
                                plsc.store_scatter(hbuf, [_rows, jcol],
                                                   col * _w)
                            return jc
                        lax.fori_loop(2 * h4, 2 * (h4 + 1), _cols, 0)

            def _step(ci, r):
                buf, gs, ss = hbufs[r], gsems[r], ssems[r]
                r2 = (r + 2) % 3
                nxt = ci + 2
                pltpu.make_async_copy(hx_hbm.at[srcg_v.at[ci]], buf, gs).wait()
                _compute(buf, ci)
                pltpu.async_copy(buf, acc.at[dst_v.at[ci]], ss, add=True)

                @pl.when(jnp.logical_and(ci >= 1, nxt < nch))
                def _wait_prev():
                    pltpu.make_async_copy(
                        hbufs[r2], acc.at[dst_v.at[0]], ssems[r2]).wait()

                @pl.when(nxt < nch)
                def _refill():
                    pltpu.async_copy(hx_hbm.at[srcg_v.at[nxt]], hbufs[r2],
                                     gsems[r2])

            pltpu.async_copy(hx_hbm.at[srcg_v.at[0]], hb0, gs0)
            pltpu.async_copy(hx_hbm.at[srcg_v.at[1]], hb1, gs1)

            def _tri(p, carry):
                for r in range(3):
                    _step(3 * p + r, r)
                return carry
            lax.fori_loop(0, nch // 3, _tri, 0)
            _step(nch - 1, (nch - 1) % 3)

            for r in range(3):
                pltpu.make_async_copy(hbufs[r], acc.at[dst_v.at[0]],
                                      ssems[r]).wait()

            plsc.subcore_barrier()

            pltpu.sync_copy(acc.at[pl.ds(r0, rows_per_tile)],
                            acc_hbm.at[pl.ds(base + r0, rows_per_tile)])
            plsc.subcore_barrier()
            return bcarry
        lax.fori_loop(0, bpc, _batch, 0)

    return sc_kernel


def kernel(x, edge_index, W, att_src, att_dst, bias):
    b, t, dm = x.shape
    n = b * t
    e = edge_index.shape[1]
    h_heads, c = att_src.shape

    x_flat = x.reshape(n, dm)
    eye = jnp.eye(h_heads, dtype=jnp.float32)
    a_src = (att_src[:, :, None] * eye[:, None, :]).reshape(dm, h_heads)
    a_dst = (att_dst[:, :, None] * eye[:, None, :]).reshape(dm, h_heads)
    amat = jnp.concatenate([a_src, a_dst], axis=1)
    sexp = jnp.repeat(eye, c, axis=1)

    blk = 1024
    grid = n // blk
    hx, d4 = pl.pallas_call(
        _pre_body,
        grid=(grid,),
        in_specs=[
            pl.BlockSpec((blk, dm), lambda i: (i, 0)),
            pl.BlockSpec((dm, dm), lambda i: (0, 0)),
            pl.BlockSpec((dm, 2 * h_heads), lambda i: (0, 0)),
        ],
        out_specs=[
            pl.BlockSpec((blk, _RW), lambda i: (i, 0)),
            pl.BlockSpec((blk, h_heads), lambda i: (i, 0)),
        ],
        out_shape=[
            jax.ShapeDtypeStruct((n, _RW), jnp.float32),
            jax.ShapeDtypeStruct((n, h_heads), jnp.float32),
        ],
    )(x_flat, W, amat)

    srcm = edge_index[0].reshape(e // _CH, _CH)
    dstm = edge_index[1].reshape(e // _CH, _CH)
    dflat = d4.reshape(n * h_heads)

    sc_fn = _make_sc_kernel(n, t, dm, e, b)
    accd = sc_fn(srcm, dstm, hx, dflat)

    out = pl.pallas_call(
        _post_body,
        grid=(grid,),
        in_specs=[
            pl.BlockSpec((blk, _RW), lambda i: (i, 0)),
            pl.BlockSpec((blk, _RW), lambda i: (i, 0)),
            pl.BlockSpec((h_heads, dm), lambda i: (0, 0)),
            pl.BlockSpec((dm,), lambda i: (0,)),
        ],
        out_specs=pl.BlockSpec((blk, dm), lambda i: (i, 0)),
        out_shape=jax.ShapeDtypeStruct((n, dm), jnp.float32),
    )(accd, hx, sexp, bias)

    return out.reshape(b, t, dm)


# speedup vs baseline: 3.6599x; 3.6599x over previous
import functools

import jax
import jax.numpy as jnp
from jax import lax
from jax.experimental import pallas as pl
from jax.experimental.pallas import tpu as pltpu
from jax.experimental.pallas import tpu_sc as plsc

_NS = 16
_NC = 2
_CH = 128
_LANES = 16
_RW = 144


def _pre_body(x_ref, w_ref, a_ref, hx_ref, d4_ref):
    h = jnp.dot(x_ref[...], w_ref[...], preferred_element_type=jnp.float32)
    sd = jnp.dot(h, a_ref[...], preferred_element_type=jnp.float32)
    pad = jnp.zeros((h.shape[0], _RW - h.shape[1] - sd.shape[1]), jnp.float32)
    hx_ref[...] = jnp.concatenate([h, sd, pad], axis=1)
    d4_ref[...] = sd[:, 4:8]


def _post_body(acc_ref, hx_ref, sexp_ref, bias_ref, out_ref):
    num = acc_ref[:, 0:128]
    den = acc_ref[:, 128:132]
    h = hx_ref[:, 0:128]
    s = hx_ref[:, 128:132]
    d = hx_ref[:, 132:136]
    a = s + d
    ws = jnp.exp(jnp.maximum(a, 0.2 * a))
    den4 = den + ws
    wsb = jnp.dot(ws, sexp_ref[...], preferred_element_type=jnp.float32)
    denb = jnp.dot(den4, sexp_ref[...], preferred_element_type=jnp.float32)
    out_ref[...] = (num + wsb * h) / (denb + 1e-16) + bias_ref[...]


def _make_sc_kernel(n, t, d, e, nbatch):
    ept = e // _NS
    nch = ept // _CH
    rows_per_tile = t // _NS
    bpc = nbatch // _NC
    mesh = plsc.VectorSubcoreMesh(core_axis_name="c", subcore_axis_name="s")

    @functools.partial(
        pl.kernel,
        out_type=jax.ShapeDtypeStruct((n, _RW), jnp.float32),
        mesh=mesh,
        scratch_types=(
            pltpu.VMEM((t * 4,), jnp.float32),
            pltpu.VMEM((nch, _CH), jnp.int32),
            pltpu.VMEM((nch, _CH), jnp.int32),
            pltpu.VMEM((_CH, _RW), jnp.float32),
            pltpu.VMEM((_CH, _RW), jnp.float32),
            pltpu.VMEM((_CH, _RW), jnp.float32),
            pltpu.VMEM_SHARED((t, _RW), jnp.float32),
            pltpu.SemaphoreType.DMA,
            pltpu.SemaphoreType.DMA,
            pltpu.SemaphoreType.DMA,
            pltpu.SemaphoreType.DMA,
            pltpu.SemaphoreType.DMA,
            pltpu.SemaphoreType.DMA,
        ),
        compiler_params=pltpu.CompilerParams(
            needs_layout_passes=False, use_tc_tiling_on_sc=False),
    )
    def sc_kernel(srcm_hbm, dstm_hbm, hx_hbm, dflat_hbm, acc_hbm,
                  sdv, srcg_v, dst_v, hb0, hb1, hb2, acc,
                  gs0, gs1, gs2, ss0, ss1, ss2):
        cid = lax.axis_index("c")
        sid = lax.axis_index("s")
        row0 = sid * nch
        r0 = sid * rows_per_tile
        ilane = lax.iota(jnp.int32, _LANES)
        izero = ilane * 0
        zero16 = izero.astype(jnp.float32)

        pltpu.sync_copy(srcm_hbm.at[pl.ds(row0, nch)], srcg_v)
        pltpu.sync_copy(dstm_hbm.at[pl.ds(row0, nch)], dst_v)
        hbufs = (hb0, hb1, hb2)
        gsems = (gs0, gs1, gs2)
        ssems = (ss0, ss1, ss2)

        def _batch(bi, bcarry):
            b = cid * bpc + bi
            base = b * t

            def _z_hb(r, carry):
                for v in range(_RW // _LANES):
                    hb0[r, pl.ds(v * _LANES, _LANES)] = zero16
                return carry
            lax.fori_loop(0, _CH, _z_hb, 0)
            for half in range(rows_per_tile // _CH):
                pltpu.sync_copy(hb0, acc.at[pl.ds(r0 + half * _CH, _CH)])

            pltpu.sync_copy(dflat_hbm.at[pl.ds(base * 4, t * 4)], sdv)

            delta = jnp.where(bi == 0, base, t)
            def _off(r, carry):
                for g in range(_CH // _LANES):
                    sl = pl.ds(g * _LANES, _LANES)
                    srcg_v[r, sl] = srcg_v[r, sl] + delta
                return carry
            lax.fori_loop(0, nch, _off, 0)

            plsc.subcore_barrier()

            def _compute(hbuf, ci):
                @plsc.parallel_loop(0, _CH // _LANES)
                def _grp(g):
                    off = g * _LANES
                    d16 = dst_v[ci, pl.ds(off, _LANES)]
                    for l in range(_LANES):
                        kk = off + l
                        dv = plsc.load_gather(sdv, [d16[l] * 4 + ilane])
                        srow = hbuf[kk, pl.ds(128, _LANES)]
                        aa = srow + dv
                        w = jnp.exp(jnp.maximum(aa, 0.2 * aa))
                        w = jnp.where(ilane < 4, w, 0.0)
                        hbuf[kk, pl.ds(128, _LANES)] = w
                        for h4 in range(4):
                            wv = jnp.full((_LANES,), w[h4], jnp.float32)
                            for v2 in range(2):
                                sl = pl.ds(h4 * 32 + v2 * _LANES, _LANES)
                                hbuf[kk, sl] = hbuf[kk, sl] * wv

            def _step(ci, r):
                buf, gs, ss = hbufs[r], gsems[r], ssems[r]
                r2 = (r + 2) % 3
                nxt = ci + 2
                pltpu.make_async_copy(hx_hbm.at[srcg_v.at[ci]], buf, gs).wait()
                _compute(buf, ci)
                pltpu.async_copy(buf, acc.at[dst_v.at[ci]], ss, add=True)

                @pl.when(jnp.logical_and(ci >= 1, nxt < nch))
                def _wait_prev():
                    pltpu.make_async_copy(
                        hbufs[r2], acc.at[dst_v.at[0]], ssems[r2]).wait()

                @pl.when(nxt < nch)
                def _refill():
                    pltpu.async_copy(hx_hbm.at[srcg_v.at[nxt]], hbufs[r2],
                                     gsems[r2])

            pltpu.async_copy(hx_hbm.at[srcg_v.at[0]], hb0, gs0)
            pltpu.async_copy(hx_hbm.at[srcg_v.at[1]], hb1, gs1)

            def _tri(p, carry):
                for r in range(3):
                    _step(3 * p + r, r)
                return carry
            lax.fori_loop(0, nch // 3, _tri, 0)
            _step(nch - 1, (nch - 1) % 3)

            for r in range(3):
                pltpu.make_async_copy(hbufs[r], acc.at[dst_v.at[0]],
                                      ssems[r]).wait()

            plsc.subcore_barrier()

            pltpu.sync_copy(acc.at[pl.ds(r0, rows_per_tile)],
                            acc_hbm.at[pl.ds(base + r0, rows_per_tile)])
            plsc.subcore_barrier()
            return bcarry
        lax.fori_loop(0, bpc, _batch, 0)

    return sc_kernel


def kernel(x, edge_index, W, att_src, att_dst, bias):
    b, t, dm = x.shape
    n = b * t
    e = edge_index.shape[1]
    h_heads, c = att_src.shape

    x_flat = x.reshape(n, dm)
    eye = jnp.eye(h_heads, dtype=jnp.float32)
    a_src = (att_src[:, :, None] * eye[:, None, :]).reshape(dm, h_heads)
    a_dst = (att_dst[:, :, None] * eye[:, None, :]).reshape(dm, h_heads)
    amat = jnp.concatenate([a_src, a_dst], axis=1)
    sexp = jnp.repeat(eye, c, axis=1)

    blk = 1024
    grid = n // blk
    hx, d4 = pl.pallas_call(
        _pre_body,
        grid=(grid,),
        in_specs=[
            pl.BlockSpec((blk, dm), lambda i: (i, 0)),
            pl.BlockSpec((dm, dm), lambda i: (0, 0)),
            pl.BlockSpec((dm, 2 * h_heads), lambda i: (0, 0)),
        ],
        out_specs=[
            pl.BlockSpec((blk, _RW), lambda i: (i, 0)),
            pl.BlockSpec((blk, h_heads), lambda i: (i, 0)),
        ],
        out_shape=[
            jax.ShapeDtypeStruct((n, _RW), jnp.float32),
            jax.ShapeDtypeStruct((n, h_heads), jnp.float32),
        ],
    )(x_flat, W, amat)

    srcm = edge_index[0].reshape(e // _CH, _CH)
    dstm = edge_index[1].reshape(e // _CH, _CH)
    dflat = d4.reshape(n * h_heads)

    sc_fn = _make_sc_kernel(n, t, dm, e, b)
    accd = sc_fn(srcm, dstm, hx, dflat)

    out = pl.pallas_call(
        _post_body,
        grid=(grid,),
        in_specs=[
            pl.BlockSpec((blk, _RW), lambda i: (i, 0)),
            pl.BlockSpec((blk, _RW), lambda i: (i, 0)),
            pl.BlockSpec((h_heads, dm), lambda i: (0, 0)),
            pl.BlockSpec((dm,), lambda i: (0,)),
        ],
        out_specs=pl.BlockSpec((blk, dm), lambda i: (i, 0)),
        out_shape=jax.ShapeDtypeStruct((n, dm), jnp.float32),
    )(accd, hx, sexp, bias)

    return out.reshape(b, t, dm)


# speedup vs baseline: 3.7137x; 1.0147x over previous
import functools

import jax
import jax.numpy as jnp
from jax import lax
from jax.experimental import pallas as pl
from jax.experimental.pallas import tpu as pltpu
from jax.experimental.pallas import tpu_sc as plsc

_NS = 16
_NC = 2
_CH = 128
_LANES = 16
_RW = 144


def _pre_body(x_ref, w_ref, a_ref, hx_ref, d4_ref):
    h = jnp.dot(x_ref[...], w_ref[...], preferred_element_type=jnp.float32)
    sd = jnp.dot(h, a_ref[...], preferred_element_type=jnp.float32)
    pad = jnp.zeros((h.shape[0], _RW - h.shape[1] - sd.shape[1]), jnp.float32)
    hx_ref[...] = jnp.concatenate([h, sd, pad], axis=1)
    d4_ref[...] = sd[:, 4:8]


def _post_body(acc_ref, hx_ref, sexp_ref, bias_ref, out_ref):
    num = acc_ref[:, 0:128]
    den = acc_ref[:, 128:132]
    h = hx_ref[:, 0:128]
    s = hx_ref[:, 128:132]
    d = hx_ref[:, 132:136]
    a = s + d
    ws = jnp.exp(jnp.maximum(a, 0.2 * a))
    den4 = den + ws
    wsb = jnp.dot(ws, sexp_ref[...], preferred_element_type=jnp.float32)
    denb = jnp.dot(den4, sexp_ref[...], preferred_element_type=jnp.float32)
    out_ref[...] = (num + wsb * h) / (denb + 1e-16) + bias_ref[...]


def _make_sc_kernel(n, t, d, e, nbatch):
    ept = e // _NS
    nch = ept // _CH
    rows_per_tile = t // _NS
    bpc = nbatch // _NC
    mesh = plsc.VectorSubcoreMesh(core_axis_name="c", subcore_axis_name="s")

    @functools.partial(
        pl.kernel,
        out_type=jax.ShapeDtypeStruct((n, _RW), jnp.float32),
        mesh=mesh,
        scratch_types=(
            pltpu.VMEM((t * 4,), jnp.float32),
            pltpu.VMEM((nch, _CH), jnp.int32),
            pltpu.VMEM((nch, _CH), jnp.int32),
            pltpu.VMEM((_CH, _RW), jnp.float32),
            pltpu.VMEM((_CH, _RW), jnp.float32),
            pltpu.VMEM((_CH, _RW), jnp.float32),
            pltpu.VMEM_SHARED((t, _RW), jnp.float32),
            pltpu.SemaphoreType.DMA,
            pltpu.SemaphoreType.DMA,
            pltpu.SemaphoreType.DMA,
            pltpu.SemaphoreType.DMA,
            pltpu.SemaphoreType.DMA,
            pltpu.SemaphoreType.DMA,
        ),
        compiler_params=pltpu.CompilerParams(
            needs_layout_passes=False, use_tc_tiling_on_sc=False),
    )
    def sc_kernel(srcm_hbm, dstm_hbm, hx_hbm, dflat_hbm, acc_hbm,
                  sdv, srcg_v, dst_v, hb0, hb1, hb2, acc,
                  gs0, gs1, gs2, ss0, ss1, ss2):
        cid = lax.axis_index("c")
        sid = lax.axis_index("s")
        row0 = sid * nch
        r0 = sid * rows_per_tile
        ilane = lax.iota(jnp.int32, _LANES)
        izero = ilane * 0
        zero16 = izero.astype(jnp.float32)

        pltpu.sync_copy(srcm_hbm.at[pl.ds(row0, nch)], srcg_v)
        pltpu.sync_copy(dstm_hbm.at[pl.ds(row0, nch)], dst_v)
        hbufs = (hb0, hb1, hb2)
        gsems = (gs0, gs1, gs2)
        ssems = (ss0, ss1, ss2)

        def _batch(bi, bcarry):
            b = cid * bpc + bi
            base = b * t

            def _z_hb(r, carry):
                for v in range(_RW // _LANES):
                    hb0[r, pl.ds(v * _LANES, _LANES)] = zero16
                return carry
            lax.fori_loop(0, _CH, _z_hb, 0)
            for half in range(rows_per_tile // _CH):
                pltpu.sync_copy(hb0, acc.at[pl.ds(r0 + half * _CH, _CH)])

            pltpu.sync_copy(dflat_hbm.at[pl.ds(base * 4, t * 4)], sdv)

            delta = jnp.where(bi == 0, base, t)
            def _off(r, carry):
                for g in range(_CH // _LANES):
                    sl = pl.ds(g * _LANES, _LANES)
                    srcg_v[r, sl] = srcg_v[r, sl] + delta
                return carry
            lax.fori_loop(0, nch, _off, 0)

            plsc.subcore_barrier()

            def _compute(hbuf, ci):
                @plsc.parallel_loop(0, _CH // _LANES)
                def _grp(g):
                    off = g * _LANES
                    d16 = dst_v[ci, pl.ds(off, _LANES)]
                    for l in range(_LANES):
                        kk = off + l
                        dv = plsc.load_gather(sdv, [d16[l] * 4 + ilane])
                        srow = hbuf[kk, pl.ds(128, _LANES)]
                        aa = srow + dv
                        w = jnp.exp(jnp.maximum(aa, 0.2 * aa))
                        hbuf[kk, pl.ds(128, _LANES)] = w
                        for h4 in range(4):
                            wv = jnp.full((_LANES,), w[h4], jnp.float32)
                            for v2 in range(2):
                                sl = pl.ds(h4 * 32 + v2 * _LANES, _LANES)
                                hbuf[kk, sl] = hbuf[kk, sl] * wv

            def _step(ci, r):
                buf, gs, ss = hbufs[r], gsems[r], ssems[r]
                r2 = (r + 2) % 3
                nxt = ci + 2
                pltpu.make_async_copy(hx_hbm.at[srcg_v.at[ci]], buf, gs).wait()
                _compute(buf, ci)
                pltpu.async_copy(buf, acc.at[dst_v.at[ci]], ss, add=True)

                @pl.when(jnp.logical_and(ci >= 1, nxt < nch))
                def _wait_prev():
                    pltpu.make_async_copy(
                        hbufs[r2], acc.at[dst_v.at[0]], ssems[r2]).wait()

                @pl.when(nxt < nch)
                def _refill():
                    pltpu.async_copy(hx_hbm.at[srcg_v.at[nxt]], hbufs[r2],
                                     gsems[r2])

            pltpu.async_copy(hx_hbm.at[srcg_v.at[0]], hb0, gs0)
            pltpu.async_copy(hx_hbm.at[srcg_v.at[1]], hb1, gs1)

            def _tri(p, carry):
                for r in range(3):
                    _step(3 * p + r, r)
                return carry
            lax.fori_loop(0, nch // 3, _tri, 0)
            _step(nch - 1, (nch - 1) % 3)

            for r in range(3):
                pltpu.make_async_copy(hbufs[r], acc.at[dst_v.at[0]],
                                      ssems[r]).wait()

            plsc.subcore_barrier()

            pltpu.sync_copy(acc.at[pl.ds(r0, rows_per_tile)],
                            acc_hbm.at[pl.ds(base + r0, rows_per_tile)])
            plsc.subcore_barrier()
            return bcarry
        lax.fori_loop(0, bpc, _batch, 0)

    return sc_kernel


def kernel(x, edge_index, W, att_src, att_dst, bias):
    b, t, dm = x.shape
    n = b * t
    e = edge_index.shape[1]
    h_heads, c = att_src.shape

    x_flat = x.reshape(n, dm)
    eye = jnp.eye(h_heads, dtype=jnp.float32)
    a_src = (att_src[:, :, None] * eye[:, None, :]).reshape(dm, h_heads)
    a_dst = (att_dst[:, :, None] * eye[:, None, :]).reshape(dm, h_heads)
    amat = jnp.concatenate([a_src, a_dst], axis=1)
    sexp = jnp.repeat(eye, c, axis=1)

    blk = 1024
    grid = n // blk
    hx, d4 = pl.pallas_call(
        _pre_body,
        grid=(grid,),
        in_specs=[
            pl.BlockSpec((blk, dm), lambda i: (i, 0)),
            pl.BlockSpec((dm, dm), lambda i: (0, 0)),
            pl.BlockSpec((dm, 2 * h_heads), lambda i: (0, 0)),
        ],
        out_specs=[
            pl.BlockSpec((blk, _RW), lambda i: (i, 0)),
            pl.BlockSpec((blk, h_heads), lambda i: (i, 0)),
        ],
        out_shape=[
            jax.ShapeDtypeStruct((n, _RW), jnp.float32),
            jax.ShapeDtypeStruct((n, h_heads), jnp.float32),
        ],
    )(x_flat, W, amat)

    srcm = edge_index[0].reshape(e // _CH, _CH)
    dstm = edge_index[1].reshape(e // _CH, _CH)
    dflat = d4.reshape(n * h_heads)

    sc_fn = _make_sc_kernel(n, t, dm, e, b)
    accd = sc_fn(srcm, dstm, hx, dflat)

    out = pl.pallas_call(
        _post_body,
        grid=(grid,),
        in_specs=[
            pl.BlockSpec((blk, _RW), lambda i: (i, 0)),
            pl.BlockSpec((blk, _RW), lambda i: (i, 0)),
            pl.BlockSpec((h_heads, dm), lambda i: (0, 0)),
            pl.BlockSpec((dm,), lambda i: (0,)),
        ],
        out_specs=pl.BlockSpec((blk, dm), lambda i: (i, 0)),
        out_shape=jax.ShapeDtypeStruct((n, dm), jnp.float32),
    )(accd, hx, sexp, bias)

    return out.reshape(b, t, dm)
